# baseline (speedup 1.0000x reference)
import functools

import jax
import jax.numpy as jnp
from jax.experimental import pallas as pl
from jax.experimental.pallas import tpu as pltpu

N, NFEAT, NHID, NCLASS = 10000, 128, 16, 4
BM = 200
GRID = N // BM


def _layer1_kernel(feat_ref, w1_ref, b1_ref, w2_ref, adj_ref, sup2_ref,
                   support_ref):
    @pl.when(pl.program_id(0) == 0)
    def _():
        support_ref[...] = jnp.dot(
            feat_ref[...], w1_ref[...],
            preferred_element_type=jnp.float32,
            precision=jax.lax.Precision.HIGHEST)

    z = jnp.dot(adj_ref[...], support_ref[...],
                preferred_element_type=jnp.float32,
                precision=jax.lax.Precision.HIGHEST)
    x1 = jnp.maximum(z + b1_ref[...], 0.0)
    sup2_ref[...] = jnp.dot(x1, w2_ref[...],
                            preferred_element_type=jnp.float32,
                            precision=jax.lax.Precision.HIGHEST)


def _layer2_kernel(sup2_ref, b2_ref, adj_ref, out_ref):
    z = jnp.dot(adj_ref[...], sup2_ref[...],
                preferred_element_type=jnp.float32,
                precision=jax.lax.Precision.HIGHEST)
    z = z + b2_ref[...]
    m = jnp.max(z, axis=1, keepdims=True)
    s = z - m
    out_ref[...] = s - jnp.log(jnp.sum(jnp.exp(s), axis=1, keepdims=True))


@jax.jit
def kernel(features, adjs, W1, b1, W2, b2):
    b1 = b1.reshape(1, NHID)
    b2 = b2.reshape(1, NCLASS)

    support2 = pl.pallas_call(
        _layer1_kernel,
        grid=(GRID,),
        in_specs=[
            pl.BlockSpec((N, NFEAT), lambda i: (0, 0)),
            pl.BlockSpec((NFEAT, NHID), lambda i: (0, 0)),
            pl.BlockSpec((1, NHID), lambda i: (0, 0)),
            pl.BlockSpec((NHID, NCLASS), lambda i: (0, 0)),
            pl.BlockSpec((BM, N), lambda i: (i, 0)),
        ],
        out_specs=pl.BlockSpec((BM, NCLASS), lambda i: (i, 0)),
        out_shape=jax.ShapeDtypeStruct((N, NCLASS), jnp.float32),
        scratch_shapes=[pltpu.VMEM((N, NHID), jnp.float32)],
        compiler_params=pltpu.CompilerParams(
            dimension_semantics=("arbitrary",)),
    )(features, W1, b1, W2, adjs)

    out = pl.pallas_call(
        _layer2_kernel,
        grid=(GRID,),
        in_specs=[
            pl.BlockSpec((N, NCLASS), lambda i: (0, 0)),
            pl.BlockSpec((1, NCLASS), lambda i: (0, 0)),
            pl.BlockSpec((BM, N), lambda i: (i, 0)),
        ],
        out_specs=pl.BlockSpec((BM, NCLASS), lambda i: (i, 0)),
        out_shape=jax.ShapeDtypeStruct((N, NCLASS), jnp.float32),
        compiler_params=pltpu.CompilerParams(
            dimension_semantics=("arbitrary",)),
    )(support2, b2, adjs)

    return out


# speedup vs baseline: 2.8799x; 2.8799x over previous
import jax
import jax.numpy as jnp
from jax.experimental import pallas as pl
from jax.experimental.pallas import tpu as pltpu

N, NFEAT, NHID, NCLASS = 10000, 128, 16, 4
BM = 400
GRID = N // BM


def _layer1_kernel(feat_ref, w1_ref, b1_ref, w2_ref, adj_ref, sup2_ref,
                   support_ref):
    @pl.when(pl.program_id(0) == 0)
    def _():
        support_ref[...] = jnp.dot(
            feat_ref[...], w1_ref[...],
            preferred_element_type=jnp.float32).astype(jnp.bfloat16)

    z = jnp.dot(adj_ref[...].astype(jnp.bfloat16), support_ref[...],
                preferred_element_type=jnp.float32)
    x1 = jnp.maximum(z + b1_ref[...], 0.0)
    sup2_ref[...] = jnp.dot(x1, w2_ref[...],
                            preferred_element_type=jnp.float32
                            ).astype(jnp.bfloat16)


def _layer2_kernel(sup2_ref, b2_ref, adj_ref, out_ref):
    z = jnp.dot(adj_ref[...].astype(jnp.bfloat16), sup2_ref[...],
                preferred_element_type=jnp.float32)
    z = z + b2_ref[...]
    m = jnp.max(z, axis=1, keepdims=True)
    s = z - m
    out_ref[...] = s - jnp.log(jnp.sum(jnp.exp(s), axis=1, keepdims=True))


@jax.jit
def kernel(features, adjs, W1, b1, W2, b2):
    b1 = b1.reshape(1, NHID)
    b2 = b2.reshape(1, NCLASS)

    support2 = pl.pallas_call(
        _layer1_kernel,
        grid=(GRID,),
        in_specs=[
            pl.BlockSpec((N, NFEAT), lambda i: (0, 0)),
            pl.BlockSpec((NFEAT, NHID), lambda i: (0, 0)),
            pl.BlockSpec((1, NHID), lambda i: (0, 0)),
            pl.BlockSpec((NHID, NCLASS), lambda i: (0, 0)),
            pl.BlockSpec((BM, N), lambda i: (i, 0)),
        ],
        out_specs=pl.BlockSpec((BM, NCLASS), lambda i: (i, 0)),
        out_shape=jax.ShapeDtypeStruct((N, NCLASS), jnp.bfloat16),
        scratch_shapes=[pltpu.VMEM((N, NHID), jnp.bfloat16)],
        compiler_params=pltpu.CompilerParams(
            dimension_semantics=("arbitrary",)),
    )(features, W1, b1, W2, adjs)

    out = pl.pallas_call(
        _layer2_kernel,
        grid=(GRID,),
        in_specs=[
            pl.BlockSpec((N, NCLASS), lambda i: (0, 0)),
            pl.BlockSpec((1, NCLASS), lambda i: (0, 0)),
            pl.BlockSpec((BM, N), lambda i: (i, 0)),
        ],
        out_specs=pl.BlockSpec((BM, NCLASS), lambda i: (i, 0)),
        out_shape=jax.ShapeDtypeStruct((N, NCLASS), jnp.float32),
        compiler_params=pltpu.CompilerParams(
            dimension_semantics=("arbitrary",)),
    )(support2, b2, adjs)

    return out
